# baseline (speedup 1.0000x reference)
import functools

import jax
import jax.numpy as jnp
from jax import lax
from jax.experimental import pallas as pl
from jax.experimental.pallas import tpu as pltpu
from jax.experimental.pallas import tpu_sc as plsc

_N = 10000
_NP = 10016
_M = 2500
_E = 160000
_EP = 163840
_D = 256
_NC = 2
_NS = 16
_NT = _NC * _NS
_L = 16

_EOWN = 80
_MP = _NT * _EOWN
_VOWN = 320
_NPAD2 = _NT * _VOWN
_VPAD = _N + 8

_SEG_A = 8192
_NSEG_A = _EP // _SEG_A
_CH_A = 128
_SEG_B = 4096
_NSEG_B = _EP // _SEG_B
_CH_B = 64


def _mm_body(x_ref, w_ref, o_ref):
    o_ref[...] = lax.dot_general(
        x_ref[...], w_ref[...], (((1,), (1,)), ((), ())),
        preferred_element_type=jnp.float32)


def _linear(X, Wlin):
    return pl.pallas_call(
        _mm_body,
        grid=(25,),
        in_specs=[pl.BlockSpec((400, _D), lambda i: (i, 0)),
                  pl.BlockSpec((_D, _D), lambda i: (0, 0))],
        out_specs=pl.BlockSpec((400, _D), lambda i: (i, 0)),
        out_shape=jax.ShapeDtypeStruct((_N, _D), jnp.float32),
    )(X, Wlin)


def _splat(vec, r):
    return jnp.take(vec, jnp.full((_L,), r, jnp.int32), axis=0, mode="fill")


_SC_PARAMS = pltpu.CompilerParams(needs_layout_passes=False)


def _phase1(xp, vertex, edges, degE, w):
    mesh = plsc.VectorSubcoreMesh(core_axis_name="c", subcore_axis_name="s")

    @functools.partial(
        pl.kernel,
        out_type=jax.ShapeDtypeStruct((_MP, _D), jnp.float32),
        mesh=mesh,
        compiler_params=_SC_PARAMS,
        scratch_types=[
            pltpu.VMEM((_SEG_A,), jnp.int32),
            pltpu.VMEM((_SEG_A,), jnp.int32),
            pltpu.VMEM((_SEG_A,), jnp.int32),
            pltpu.VMEM((_SEG_A,), jnp.int32),
            pltpu.VMEM((_CH_A, _D), jnp.float32),
            pltpu.VMEM((_EOWN + 8, _D), jnp.float32),
            pltpu.VMEM((_MP,), jnp.float32),
            pltpu.VMEM((_MP,), jnp.float32),
            pltpu.VMEM((_L,), jnp.int32),
            pltpu.SMEM((_L,), jnp.int32),
            pltpu.VMEM_SHARED((_NS, _L), jnp.int32),
            pltpu.SemaphoreType.DMA,
        ],
    )
    def k(xp_hbm, v_hbm, e_hbm, degE_hbm, w_hbm, out_hbm,
          vseg, eseg, cv, cle, gbuf, acc, degE_v, w_v, cntv, cnts, cntsh,
          sem):
        c = lax.axis_index("c")
        s = lax.axis_index("s")
        t = c * _NS + s
        own0 = t * _EOWN
        iota = lax.iota(jnp.int32, _L)
        zerosf = jnp.zeros((_L,), jnp.float32)
        dumv = jnp.full((_L,), _VPAD, jnp.int32)
        dumle = jnp.full((_L,), _EOWN, jnp.int32)

        pltpu.sync_copy(degE_hbm, degE_v)
        pltpu.sync_copy(w_hbm, w_v)

        @pl.loop(0, _EOWN + 8, unroll=4)
        def _(r):
            for j in range(_D // _L):
                acc[r, pl.ds(j * _L, _L)] = zerosf

        @pl.loop(0, _NSEG_A)
        def _(seg):
            soff = seg * _SEG_A
            pltpu.sync_copy(v_hbm.at[pl.ds(soff, _SEG_A)], vseg)
            pltpu.sync_copy(e_hbm.at[pl.ds(soff, _SEG_A)], eseg)

            @pl.loop(0, _SEG_A // _L, unroll=4)
            def _(g):
                cv[pl.ds(g * _L, _L)] = dumv
                cle[pl.ds(g * _L, _L)] = dumle

            @pl.loop(0, _SEG_A // _L, init_carry=jnp.zeros((_L,), jnp.int32))
            def scan(g, ptr):
                v_vec = vseg[pl.ds(g * _L, _L)]
                rel = eseg[pl.ds(g * _L, _L)] - own0
                own = (rel >= 0) & (rel < _EOWN)
                inc = plsc.cumsum(jnp.where(own, 1, 0).astype(jnp.int32))
                pos = ptr + inc - 1
                plsc.store_scatter(cv, [pos], v_vec, mask=own)
                plsc.store_scatter(cle, [pos], rel, mask=own)
                return ptr + _splat(inc, _L - 1)

            cntv[...] = scan
            pltpu.sync_copy(cntv, cntsh.at[s])
            pltpu.sync_copy(cntsh.at[s], cnts)
            cnt = cnts[0]
            nch = jnp.minimum(jnp.maximum((cnt + _CH_A - 1) >> 7, 0),
                              _SEG_A // _CH_A)

            @pl.loop(0, nch)
            def _(i):
                off = i * _CH_A
                pltpu.async_copy(
                    xp_hbm.at[cv.at[pl.ds(off, _CH_A)]], gbuf, sem).wait()

                @pl.loop(0, _CH_A // _L)
                def _(g2):
                    le_vec = cle[pl.ds(off + g2 * _L, _L)]
                    for r in range(_L):
                        le = _splat(le_vec, r)
                        vals = [gbuf[g2 * _L + r, pl.ds(j * _L, _L)]
                                for j in range(_D // _L)]
                        for j in range(_D // _L):
                            plsc.addupdate_scatter(
                                acc, [le, iota + (j * _L)], vals[j])

        @pl.loop(0, _EOWN)
        def _(r):
            gidx = jnp.full((_L,), own0 + r, jnp.int32)
            sv = (plsc.load_gather(degE_v, [gidx])
                  * plsc.load_gather(w_v, [gidx]))
            for j in range(_D // _L):
                sl = pl.ds(j * _L, _L)
                acc[r, sl] = acc[r, sl] * sv

        pltpu.sync_copy(acc.at[pl.ds(0, _EOWN)],
                        out_hbm.at[pl.ds(own0, _EOWN)])

    return k(xp, vertex, edges, degE, w)


def _phase2(xe2, vertex, edges, degV):
    mesh = plsc.VectorSubcoreMesh(core_axis_name="c", subcore_axis_name="s")

    @functools.partial(
        pl.kernel,
        out_type=jax.ShapeDtypeStruct((_N, _D), jnp.float32),
        mesh=mesh,
        compiler_params=_SC_PARAMS,
        scratch_types=[
            pltpu.VMEM((_SEG_B,), jnp.int32),
            pltpu.VMEM((_SEG_B,), jnp.int32),
            pltpu.VMEM((_SEG_B,), jnp.int32),
            pltpu.VMEM((_SEG_B,), jnp.int32),
            pltpu.VMEM((_CH_B, _D), jnp.float32),
            pltpu.VMEM((_VOWN + 8, _D), jnp.float32),
            pltpu.VMEM((_VOWN,), jnp.float32),
            pltpu.VMEM((_L,), jnp.int32),
            pltpu.SMEM((_L,), jnp.int32),
            pltpu.VMEM_SHARED((_NS, _L), jnp.int32),
            pltpu.SemaphoreType.DMA,
        ],
    )
    def k(xe2_hbm, v_hbm, e_hbm, degV_hbm, out_hbm,
          vseg, eseg, ce, clv, gbuf, acc, degV_v, cntv, cnts, cntsh,
          sem):
        c = lax.axis_index("c")
        s = lax.axis_index("s")
        t = c * _NS + s
        own0 = t * _VOWN
        iota = lax.iota(jnp.int32, _L)
        zerosf = jnp.zeros((_L,), jnp.float32)
        dume = jnp.full((_L,), _M, jnp.int32)
        dumlv = jnp.full((_L,), _VOWN, jnp.int32)

        pltpu.sync_copy(degV_hbm.at[pl.ds(own0, _VOWN)], degV_v)

        @pl.loop(0, _VOWN + 8, unroll=4)
        def _(r):
            for j in range(_D // _L):
                acc[r, pl.ds(j * _L, _L)] = zerosf

        @pl.loop(0, _NSEG_B)
        def _(seg):
            soff = seg * _SEG_B
            pltpu.sync_copy(v_hbm.at[pl.ds(soff, _SEG_B)], vseg)
            pltpu.sync_copy(e_hbm.at[pl.ds(soff, _SEG_B)], eseg)

            @pl.loop(0, _SEG_B // _L, unroll=4)
            def _(g):
                ce[pl.ds(g * _L, _L)] = dume
                clv[pl.ds(g * _L, _L)] = dumlv

            @pl.loop(0, _SEG_B // _L, init_carry=jnp.zeros((_L,), jnp.int32))
            def scan(g, ptr):
                e_vec = eseg[pl.ds(g * _L, _L)]
                rel = vseg[pl.ds(g * _L, _L)] - own0
                own = (rel >= 0) & (rel < _VOWN)
                inc = plsc.cumsum(jnp.where(own, 1, 0).astype(jnp.int32))
                pos = ptr + inc - 1
                plsc.store_scatter(ce, [pos], e_vec, mask=own)
                plsc.store_scatter(clv, [pos], rel, mask=own)
                return ptr + _splat(inc, _L - 1)

            cntv[...] = scan
            pltpu.sync_copy(cntv, cntsh.at[s])
            pltpu.sync_copy(cntsh.at[s], cnts)
            cnt = cnts[0]
            nch = jnp.minimum(jnp.maximum((cnt + _CH_B - 1) >> 6, 0),
                              _SEG_B // _CH_B)

            @pl.loop(0, nch)
            def _(i):
                off = i * _CH_B
                pltpu.async_copy(
                    xe2_hbm.at[ce.at[pl.ds(off, _CH_B)]], gbuf, sem).wait()

                @pl.loop(0, _CH_B // _L)
                def _(g2):
                    lv_vec = clv[pl.ds(off + g2 * _L, _L)]
                    for r in range(_L):
                        lv = _splat(lv_vec, r)
                        vals = [gbuf[g2 * _L + r, pl.ds(j * _L, _L)]
                                for j in range(_D // _L)]
                        for j in range(_D // _L):
                            plsc.addupdate_scatter(
                                acc, [lv, iota + (j * _L)], vals[j])

        @pl.loop(0, _VOWN)
        def _(r):
            dv = plsc.load_gather(degV_v, [jnp.full((_L,), r, jnp.int32)])
            for j in range(_D // _L):
                sl = pl.ds(j * _L, _L)
                acc[r, sl] = acc[r, sl] * dv

        @pl.when(t < _NT - 1)
        def _():
            pltpu.sync_copy(acc.at[pl.ds(0, _VOWN)],
                            out_hbm.at[pl.ds(own0, _VOWN)])

        @pl.when(t == _NT - 1)
        def _():
            pltpu.sync_copy(acc.at[pl.ds(0, _N - (_NT - 1) * _VOWN)],
                            out_hbm.at[pl.ds((_NT - 1) * _VOWN,
                                             _N - (_NT - 1) * _VOWN)])

    return k(xe2, vertex, edges, degV)


def _tc_copy(x):
    def body(x_ref, o_ref):
        o_ref[...] = x_ref[...]
    return pl.pallas_call(
        body,
        grid=(10,),
        in_specs=[pl.BlockSpec((_MP // 10, _D), lambda i: (i, 0))],
        out_specs=pl.BlockSpec((_MP // 10, _D), lambda i: (i, 0)),
        out_shape=jax.ShapeDtypeStruct((_MP, _D), jnp.float32),
    )(x)


def kernel(X, vertex, edges, Wlin, degE, degV, W):
    Xp = _linear(X, Wlin)
    xp_pad = jnp.pad(Xp, ((0, _NP - _N), (0, 0)))
    epad = ((0, _EP - _E),)
    v32 = jnp.pad(vertex.astype(jnp.int32), epad, constant_values=_VPAD)
    e32 = jnp.pad(edges.astype(jnp.int32), epad, constant_values=_M)
    degEp = jnp.pad(degE[:, 0], ((0, _MP - _M),))
    wp = jnp.pad(W[:, 0], ((0, _MP - _M),))
    degVp = jnp.pad(degV[:, 0], ((0, _NPAD2 - _N),))
    xe2 = _tc_copy(_phase1(xp_pad, v32, e32, degEp, wp))
    return _phase2(xe2, v32, e32, degVp)
